# baseline (speedup 1.0000x reference)
import functools

import jax
import jax.numpy as jnp
from jax import lax
from jax.experimental import pallas as pl
from jax.experimental.pallas import tpu as pltpu
from jax.experimental.pallas import tpu_sc as plsc

_B, _T = 16, 16000
_N_IN, _N_OUT = 256, 512
_NTOT = _B * _T
_CH = 160
_TTT = 3200
_NT = _T // _TTT


def _make_sc_gather():
    info = plsc.get_sparse_core_info()
    nc, ns = info.num_cores, info.num_subcores
    nw = nc * ns
    n_per_w = _NTOT // nw
    n_iter = n_per_w // _CH
    mesh = plsc.VectorSubcoreMesh(core_axis_name="c", subcore_axis_name="s")

    @functools.partial(
        pl.kernel, mesh=mesh,
        out_type=jax.ShapeDtypeStruct((_NTOT, _N_OUT), jnp.float32),
        scratch_types=[
            pltpu.VMEM((_CH,), jnp.int32),
            pltpu.VMEM((_CH, _N_OUT), jnp.float32),
            pltpu.SemaphoreType.DMA,
        ],
    )
    def sc_gather(table_hbm, idx_hbm, out_hbm, idx_v, rows_v, sem):
        wid = lax.axis_index("s") * nc + lax.axis_index("c")
        base = wid * n_per_w

        @pl.loop(0, n_iter)
        def _(j):
            off = base + j * _CH
            pltpu.sync_copy(idx_hbm.at[pl.ds(off, _CH)], idx_v)
            pltpu.async_copy(table_hbm.at[idx_v], rows_v, sem).wait()
            pltpu.sync_copy(rows_v, out_hbm.at[pl.ds(off, _CH)])

    return sc_gather


def _xpose_kernel(in_ref, out_ref):
    v = in_ref[0, 0]
    v = v / (1.0 + jnp.abs(v))
    out_ref[0] = jnp.transpose(v, (1, 0))


def _transpose_softsign(y_flat):
    y4 = y_flat.reshape(_B, _NT, _TTT, _N_OUT)
    return pl.pallas_call(
        _xpose_kernel,
        grid=(_B, _NT),
        in_specs=[pl.BlockSpec((1, 1, _TTT, _N_OUT), lambda b, t: (b, t, 0, 0))],
        out_specs=pl.BlockSpec((1, _N_OUT, _TTT), lambda b, t: (b, 0, t)),
        out_shape=jax.ShapeDtypeStruct((_B, _N_OUT, _T), jnp.float32),
    )(y4)


def kernel(x, table):
    x_flat = x.astype(jnp.int32).reshape(_NTOT)
    y_flat = _make_sc_gather()(table, x_flat)
    return _transpose_softsign(y_flat)


# speedup vs baseline: 5.4075x; 5.4075x over previous
import jax
import jax.numpy as jnp
from jax.experimental import pallas as pl

_B, _T = 16, 16000
_N_IN, _N_OUT = 256, 512
_CC = 256
_C2 = _N_OUT // _CC
_TC = 3200
_NC = _T // _TC


def _onehot_kernel(x_ref, tab_ref, out_ref):
    tab = tab_ref[...]
    ss = (tab / (1.0 + jnp.abs(tab))).astype(jnp.bfloat16)
    iota = jax.lax.broadcasted_iota(jnp.int32, (_N_IN, _TC), 0)
    for n in range(_NC):
        idx = x_ref[0, 0, n * _TC:(n + 1) * _TC]
        oh = (iota == idx[None, :]).astype(jnp.bfloat16)
        out_ref[0, :, n * _TC:(n + 1) * _TC] = jax.lax.dot_general(
            ss, oh,
            (((0,), (0,)), ((), ())),
            preferred_element_type=jnp.float32,
        )


def _lookup(x, table):
    b = x.shape[0]
    x3 = x.astype(jnp.int32).reshape(b, 1, _T)
    return pl.pallas_call(
        _onehot_kernel,
        grid=(b, _C2),
        in_specs=[
            pl.BlockSpec((1, 1, _T), lambda i, c: (i, 0, 0)),
            pl.BlockSpec((_N_IN, _CC), lambda i, c: (0, c)),
        ],
        out_specs=pl.BlockSpec((1, _CC, _T), lambda i, c: (i, c, 0)),
        out_shape=jax.ShapeDtypeStruct((b, _N_OUT, _T), jnp.float32),
    )(x3, table)


def kernel(x, table):
    return _lookup(x, table)
